# Initial kernel scaffold; baseline (speedup 1.0000x reference)
import jax
import jax.numpy as jnp
from jax.experimental import pallas as pl


def kernel(x, edge_index, W1, b1, W2, b2, W3, b3):
    raise NotImplementedError("write your pallas kernel here")


# speedup vs baseline: 9.1838x; 9.1838x over previous
import functools
from functools import partial

import jax
import jax.numpy as jnp
from jax import lax
from jax.experimental import pallas as pl
from jax.experimental.pallas import tpu as pltpu
from jax.experimental.pallas import tpu_sc as plsc

NC = 2
NS = 16
NW = NC * NS
K = 128


def _sc_degree(dst_r, np_rows):
  C = dst_r.shape[1]
  rows_pt = np_rows // NS
  mesh = plsc.VectorSubcoreMesh(core_axis_name="c", subcore_axis_name="s")

  @functools.partial(
      pl.kernel,
      out_type=jax.ShapeDtypeStruct((NC, np_rows), jnp.float32),
      mesh=mesh,
      scratch_types=[
          pltpu.VMEM_SHARED((np_rows,), jnp.float32),
          pltpu.VMEM((C, K), jnp.int32),
          pltpu.VMEM((K,), jnp.float32),
          pltpu.VMEM((rows_pt,), jnp.float32),
      ],
  )
  def deg_kernel(dst_hbm, out_hbm, deg_sh, idx_v, ones_v, zero_v):
    c = lax.axis_index("c")
    s = lax.axis_index("s")
    wid = c * NS + s
    for i in range(K // 16):
      ones_v[pl.ds(i * 16, 16)] = jnp.ones((16,), jnp.float32)
    for i in range(rows_pt // 16):
      zero_v[pl.ds(i * 16, 16)] = jnp.zeros((16,), jnp.float32)
    pltpu.sync_copy(zero_v, deg_sh.at[pl.ds(s * rows_pt, rows_pt)])
    pltpu.sync_copy(dst_hbm.at[wid], idx_v)
    plsc.subcore_barrier()

    def body(j, carry):
      pltpu.sync_copy(ones_v, deg_sh.at[idx_v.at[j]], add=True)
      return carry

    lax.fori_loop(0, C, body, 0, unroll=False)
    plsc.subcore_barrier()
    pltpu.sync_copy(deg_sh.at[pl.ds(s * rows_pt, rows_pt)],
                    out_hbm.at[c, pl.ds(s * rows_pt, rows_pt)])

  return deg_kernel(dst_r)


def _sc_scatter(hp, src_r, dst_r, zeros2d, np_rows, d):
  C = src_r.shape[1]
  rows_pt = np_rows // NS
  mesh = plsc.VectorSubcoreMesh(core_axis_name="c", subcore_axis_name="s")

  @functools.partial(
      pl.kernel,
      out_type=jax.ShapeDtypeStruct((NC, np_rows, d), jnp.float32),
      mesh=mesh,
      scratch_types=[
          pltpu.VMEM_SHARED((np_rows, d), jnp.float32),
          pltpu.VMEM((C, K), jnp.int32),
          pltpu.VMEM((C, K), jnp.int32),
          pltpu.VMEM((K, d), jnp.float32),
      ],
  )
  def scat_kernel(hp_hbm, src_hbm, dst_hbm, zero_hbm, out_hbm,
                  acc_sh, src_v, dst_v, rows_v):
    c = lax.axis_index("c")
    s = lax.axis_index("s")
    wid = c * NS + s
    row_slice = pl.ds(s * rows_pt, rows_pt)

    @pl.when(c == 0)
    def _():
      pltpu.sync_copy(hp_hbm.at[row_slice], acc_sh.at[row_slice])

    @pl.when(c != 0)
    def _():
      pltpu.sync_copy(zero_hbm.at[row_slice], acc_sh.at[row_slice])

    pltpu.sync_copy(src_hbm.at[wid], src_v)
    pltpu.sync_copy(dst_hbm.at[wid], dst_v)
    plsc.subcore_barrier()

    def body(j, carry):
      pltpu.sync_copy(hp_hbm.at[src_v.at[j]], rows_v)
      pltpu.sync_copy(rows_v, acc_sh.at[dst_v.at[j]], add=True)
      return carry

    lax.fori_loop(0, C, body, 0, unroll=False)
    plsc.subcore_barrier()
    pltpu.sync_copy(acc_sh.at[row_slice], out_hbm.at[c, row_slice])

  return scat_kernel(hp, src_r, dst_r, zeros2d)


def _tc_dinv(deg2, n, np_rows):
  blocks = np_rows // 128
  deg_r = deg2.reshape(NC, blocks, 128)

  def body(deg_ref, out_ref):
    deg = deg_ref[0] + deg_ref[1] + 1.0
    rid = (lax.broadcasted_iota(jnp.int32, (blocks, 128), 0) * 128
           + lax.broadcasted_iota(jnp.int32, (blocks, 128), 1))
    out_ref[...] = jnp.where(rid < n, lax.rsqrt(deg), 0.0)

  out = pl.pallas_call(
      body,
      out_shape=jax.ShapeDtypeStruct((blocks, 128), jnp.float32),
  )(deg_r)
  return out.reshape(np_rows, 1)


def _tc_mm_scale(x, w, dinv, np_rows, d):
  def body(x_ref, w_ref, dinv_ref, out_ref):
    out_ref[...] = jnp.dot(
        x_ref[...], w_ref[...], preferred_element_type=jnp.float32
    ) * dinv_ref[...]

  return pl.pallas_call(
      body,
      grid=(np_rows // 128,),
      in_specs=[
          pl.BlockSpec((128, d), lambda i: (i, 0)),
          pl.BlockSpec((d, d), lambda i: (0, 0)),
          pl.BlockSpec((128, 1), lambda i: (i, 0)),
      ],
      out_specs=pl.BlockSpec((128, d), lambda i: (i, 0)),
      out_shape=jax.ShapeDtypeStruct((np_rows, d), jnp.float32),
  )(x, w, dinv)


def _tc_fused_next(acc, dinv, b, w_next, np_rows, d):
  def body(acc_ref, dinv_ref, b_ref, w_ref, out_ref):
    v = dinv_ref[...] * (acc_ref[0] + acc_ref[1]) + b_ref[...]
    v = jnp.maximum(v, 0.0)
    out_ref[...] = jnp.dot(
        v, w_ref[...], preferred_element_type=jnp.float32
    ) * dinv_ref[...]

  return pl.pallas_call(
      body,
      grid=(np_rows // 128,),
      in_specs=[
          pl.BlockSpec((NC, 128, d), lambda i: (0, i, 0)),
          pl.BlockSpec((128, 1), lambda i: (i, 0)),
          pl.BlockSpec((1, d), lambda i: (0, 0)),
          pl.BlockSpec((d, d), lambda i: (0, 0)),
      ],
      out_specs=pl.BlockSpec((128, d), lambda i: (i, 0)),
      out_shape=jax.ShapeDtypeStruct((np_rows, d), jnp.float32),
  )(acc, dinv, b.reshape(1, d), w_next)


def _tc_final(acc, dinv, b, n, d):
  def body(acc_ref, dinv_ref, b_ref, out_ref):
    v = dinv_ref[...] * (acc_ref[0] + acc_ref[1]) + b_ref[...]
    ss = jnp.sum(v * v, axis=-1, keepdims=True)
    out_ref[...] = v * lax.rsqrt(jnp.maximum(ss, 1e-24))

  grid = (n + 127) // 128
  return pl.pallas_call(
      body,
      grid=(grid,),
      in_specs=[
          pl.BlockSpec((NC, 128, d), lambda i: (0, i, 0)),
          pl.BlockSpec((128, 1), lambda i: (i, 0)),
          pl.BlockSpec((1, d), lambda i: (0, 0)),
      ],
      out_specs=pl.BlockSpec((128, d), lambda i: (i, 0)),
      out_shape=jax.ShapeDtypeStruct((n, d), jnp.float32),
  )(acc, dinv, b.reshape(1, d))


@jax.jit
def kernel(x, edge_index, W1, b1, W2, b2, W3, b3):
  n, d = x.shape
  e = edge_index.shape[1]
  dummy = n
  np_rows = ((n + 1 + NS * 128 - 1) // (NS * 128)) * (NS * 128)

  src = edge_index[0].astype(jnp.int32)
  dst = edge_index[1].astype(jnp.int32)
  C = (e + NW * K - 1) // (NW * K)
  e2 = NW * C * K
  pad = e2 - e
  src_r = jnp.concatenate(
      [src, jnp.full((pad,), dummy, jnp.int32)]).reshape(NW, C, K)
  dst_r = jnp.concatenate(
      [dst, jnp.full((pad,), dummy, jnp.int32)]).reshape(NW, C, K)

  x_pad = jnp.zeros((np_rows, d), jnp.float32).at[:n].set(x)
  zeros2d = jnp.zeros((np_rows, d), jnp.float32)

  deg2 = _sc_degree(dst_r, np_rows)
  dinv = _tc_dinv(deg2, n, np_rows)

  hp1 = _tc_mm_scale(x_pad, W1, dinv, np_rows, d)
  acc1 = _sc_scatter(hp1, src_r, dst_r, zeros2d, np_rows, d)
  hp2 = _tc_fused_next(acc1, dinv, b1, W2, np_rows, d)
  acc2 = _sc_scatter(hp2, src_r, dst_r, zeros2d, np_rows, d)
  hp3 = _tc_fused_next(acc2, dinv, b2, W3, np_rows, d)
  acc3 = _sc_scatter(hp3, src_r, dst_r, zeros2d, np_rows, d)
  return _tc_final(acc3, dinv, b3, n, d)


# speedup vs baseline: 19.0329x; 2.0724x over previous
import functools
from functools import partial

import jax
import jax.numpy as jnp
from jax import lax
from jax.experimental import pallas as pl
from jax.experimental.pallas import tpu as pltpu
from jax.experimental.pallas import tpu_sc as plsc

NC = 2
NS = 16
NW = NC * NS
K = 128


def _sc_degree(dst_r, np_rows):
  C = dst_r.shape[1]
  rows_pt = np_rows // NS
  mesh = plsc.VectorSubcoreMesh(core_axis_name="c", subcore_axis_name="s")

  @functools.partial(
      pl.kernel,
      out_type=jax.ShapeDtypeStruct((NC, np_rows), jnp.float32),
      mesh=mesh,
      scratch_types=[
          pltpu.VMEM_SHARED((np_rows,), jnp.float32),
          pltpu.VMEM((C, K), jnp.int32),
          pltpu.VMEM((K,), jnp.float32),
          pltpu.VMEM((rows_pt,), jnp.float32),
      ],
  )
  def deg_kernel(dst_hbm, out_hbm, deg_sh, idx_v, ones_v, zero_v):
    c = lax.axis_index("c")
    s = lax.axis_index("s")
    wid = c * NS + s
    for i in range(K // 16):
      ones_v[pl.ds(i * 16, 16)] = jnp.ones((16,), jnp.float32)
    for i in range(rows_pt // 16):
      zero_v[pl.ds(i * 16, 16)] = jnp.zeros((16,), jnp.float32)
    pltpu.sync_copy(zero_v, deg_sh.at[pl.ds(s * rows_pt, rows_pt)])
    pltpu.sync_copy(dst_hbm.at[wid], idx_v)
    plsc.subcore_barrier()

    def body(j, carry):
      pltpu.sync_copy(ones_v, deg_sh.at[idx_v.at[j]], add=True)
      return carry

    lax.fori_loop(0, C, body, 0, unroll=False)
    plsc.subcore_barrier()
    pltpu.sync_copy(deg_sh.at[pl.ds(s * rows_pt, rows_pt)],
                    out_hbm.at[c, pl.ds(s * rows_pt, rows_pt)])

  return deg_kernel(dst_r)


def _sc_scatter(hp, src_r, dst_r, zeros2d, np_rows, d):
  C = src_r.shape[1]
  rows_pt = np_rows // NS
  mesh = plsc.VectorSubcoreMesh(core_axis_name="c", subcore_axis_name="s")

  @functools.partial(
      pl.kernel,
      out_type=jax.ShapeDtypeStruct((NC, np_rows, d), jnp.float32),
      mesh=mesh,
      scratch_types=[
          pltpu.VMEM_SHARED((np_rows, d), jnp.float32),
          pltpu.VMEM((C, K), jnp.int32),
          pltpu.VMEM((4, K), jnp.int32),
          pltpu.VMEM((2, K, d), jnp.float32),
          pltpu.SemaphoreType.DMA,
          pltpu.SemaphoreType.DMA,
          pltpu.SemaphoreType.DMA,
          pltpu.SemaphoreType.DMA,
          [pltpu.SemaphoreType.DMA] * 4,
      ],
  )
  def scat_kernel(hp_hbm, src_hbm, dst_hbm, zero_hbm, out_hbm,
                  acc_sh, src_v, didx_v, rows_v, g0, g1, s0, s1, dsems):
    c = lax.axis_index("c")
    s = lax.axis_index("s")
    wid = c * NS + s
    row_slice = pl.ds(s * rows_pt, rows_pt)

    @pl.when(c == 0)
    def _():
      pltpu.sync_copy(hp_hbm.at[row_slice], acc_sh.at[row_slice])

    @pl.when(c != 0)
    def _():
      pltpu.sync_copy(zero_hbm.at[row_slice], acc_sh.at[row_slice])

    pltpu.sync_copy(src_hbm.at[wid], src_v)
    plsc.subcore_barrier()

    gsems = (g0, g1)
    ssems = (s0, s1)

    def wait_gather(p):
      pltpu.make_async_copy(
          hp_hbm.at[src_v.at[0]], rows_v.at[p], gsems[p]).wait()

    def wait_scatter(p):
      pltpu.make_async_copy(
          rows_v.at[0], acc_sh.at[didx_v.at[0]], ssems[p]).wait()

    def wait_didx(q):
      pltpu.make_async_copy(
          dst_hbm.at[wid, 0], didx_v.at[q], dsems[q]).wait()

    for q in range(3):
      pltpu.async_copy(dst_hbm.at[wid, q], didx_v.at[q], dsems[q])
    pltpu.async_copy(hp_hbm.at[src_v.at[0]], rows_v.at[0], g0)

    @pl.loop(0, C, step=4)
    def _(j):
      for u in range(4):
        jj = j + u
        p = u % 2
        wait_gather(p)
        wait_didx(u)
        pltpu.async_copy(rows_v.at[p], acc_sh.at[didx_v.at[u]], ssems[p],
                         add=True)
        if u == 0:
          @pl.when(j > 0)
          def _():
            wait_scatter(1)
        else:
          wait_scatter(1 - p)
        nxt = jj + 1
        if u < 3:
          pltpu.async_copy(hp_hbm.at[src_v.at[nxt]], rows_v.at[1 - p],
                           gsems[1 - p])
        else:
          @pl.when(nxt < C)
          def _():
            pltpu.async_copy(hp_hbm.at[src_v.at[nxt]], rows_v.at[1 - p],
                             gsems[1 - p])
        nid = jj + 3

        @pl.when(nid < C)
        def _():
          pltpu.async_copy(dst_hbm.at[wid, nid], didx_v.at[(u + 3) % 4],
                           dsems[(u + 3) % 4])

    wait_scatter((C - 1) % 2)
    plsc.subcore_barrier()
    pltpu.sync_copy(acc_sh.at[row_slice], out_hbm.at[c, row_slice])

  return scat_kernel(hp, src_r, dst_r, zeros2d)


def _tc_dinv(deg2, n, np_rows):
  blocks = np_rows // 128
  deg_r = deg2.reshape(NC, blocks, 128)

  def body(deg_ref, out_ref):
    deg = deg_ref[0] + deg_ref[1] + 1.0
    rid = (lax.broadcasted_iota(jnp.int32, (blocks, 128), 0) * 128
           + lax.broadcasted_iota(jnp.int32, (blocks, 128), 1))
    out_ref[...] = jnp.where(rid < n, lax.rsqrt(deg), 0.0)

  out = pl.pallas_call(
      body,
      out_shape=jax.ShapeDtypeStruct((blocks, 128), jnp.float32),
  )(deg_r)
  return out.reshape(np_rows, 1)


def _tc_mm_scale(x, w, dinv, np_rows, d):
  def body(x_ref, w_ref, dinv_ref, out_ref):
    out_ref[...] = jnp.dot(
        x_ref[...], w_ref[...], preferred_element_type=jnp.float32
    ) * dinv_ref[...]

  return pl.pallas_call(
      body,
      grid=(np_rows // 128,),
      in_specs=[
          pl.BlockSpec((128, d), lambda i: (i, 0)),
          pl.BlockSpec((d, d), lambda i: (0, 0)),
          pl.BlockSpec((128, 1), lambda i: (i, 0)),
      ],
      out_specs=pl.BlockSpec((128, d), lambda i: (i, 0)),
      out_shape=jax.ShapeDtypeStruct((np_rows, d), jnp.float32),
  )(x, w, dinv)


def _tc_fused_next(acc, dinv, b, w_next, np_rows, d):
  def body(acc_ref, dinv_ref, b_ref, w_ref, out_ref):
    v = dinv_ref[...] * (acc_ref[0] + acc_ref[1]) + b_ref[...]
    v = jnp.maximum(v, 0.0)
    out_ref[...] = jnp.dot(
        v, w_ref[...], preferred_element_type=jnp.float32
    ) * dinv_ref[...]

  return pl.pallas_call(
      body,
      grid=(np_rows // 128,),
      in_specs=[
          pl.BlockSpec((NC, 128, d), lambda i: (0, i, 0)),
          pl.BlockSpec((128, 1), lambda i: (i, 0)),
          pl.BlockSpec((1, d), lambda i: (0, 0)),
          pl.BlockSpec((d, d), lambda i: (0, 0)),
      ],
      out_specs=pl.BlockSpec((128, d), lambda i: (i, 0)),
      out_shape=jax.ShapeDtypeStruct((np_rows, d), jnp.float32),
  )(acc, dinv, b.reshape(1, d), w_next)


def _tc_final(acc, dinv, b, n, d):
  def body(acc_ref, dinv_ref, b_ref, out_ref):
    v = dinv_ref[...] * (acc_ref[0] + acc_ref[1]) + b_ref[...]
    ss = jnp.sum(v * v, axis=-1, keepdims=True)
    out_ref[...] = v * lax.rsqrt(jnp.maximum(ss, 1e-24))

  grid = (n + 127) // 128
  return pl.pallas_call(
      body,
      grid=(grid,),
      in_specs=[
          pl.BlockSpec((NC, 128, d), lambda i: (0, i, 0)),
          pl.BlockSpec((128, 1), lambda i: (i, 0)),
          pl.BlockSpec((1, d), lambda i: (0, 0)),
      ],
      out_specs=pl.BlockSpec((128, d), lambda i: (i, 0)),
      out_shape=jax.ShapeDtypeStruct((n, d), jnp.float32),
  )(acc, dinv, b.reshape(1, d))


@jax.jit
def kernel(x, edge_index, W1, b1, W2, b2, W3, b3):
  n, d = x.shape
  e = edge_index.shape[1]
  dummy = n
  np_rows = ((n + 1 + NS * 128 - 1) // (NS * 128)) * (NS * 128)

  src = edge_index[0].astype(jnp.int32)
  dst = edge_index[1].astype(jnp.int32)
  C = (e + NW * K - 1) // (NW * K)
  C = ((C + 3) // 4) * 4
  e2 = NW * C * K
  pad = e2 - e
  fill = dummy + (jnp.arange(pad, dtype=jnp.int32) % (np_rows - n))
  src_r = jnp.concatenate([src, fill]).reshape(NW, C, K)
  dst_r = jnp.concatenate([dst, fill]).reshape(NW, C, K)

  x_pad = jnp.zeros((np_rows, d), jnp.float32).at[:n].set(x)
  zeros2d = jnp.zeros((np_rows, d), jnp.float32)

  deg2 = _sc_degree(dst_r, np_rows)
  dinv = _tc_dinv(deg2, n, np_rows)

  hp1 = _tc_mm_scale(x_pad, W1, dinv, np_rows, d)
  acc1 = _sc_scatter(hp1, src_r, dst_r, zeros2d, np_rows, d)
  hp2 = _tc_fused_next(acc1, dinv, b1, W2, np_rows, d)
  acc2 = _sc_scatter(hp2, src_r, dst_r, zeros2d, np_rows, d)
  hp3 = _tc_fused_next(acc2, dinv, b2, W3, np_rows, d)
  acc3 = _sc_scatter(hp3, src_r, dst_r, zeros2d, np_rows, d)
  return _tc_final(acc3, dinv, b3, n, d)


# speedup vs baseline: 21.3868x; 1.1237x over previous
import functools
from functools import partial

import jax
import jax.numpy as jnp
from jax import lax
from jax.experimental import pallas as pl
from jax.experimental.pallas import tpu as pltpu
from jax.experimental.pallas import tpu_sc as plsc

NC = 2
NS = 16
NW = NC * NS
K = 128


def _sc_degree(dst_r, np_rows):
  C = dst_r.shape[1]
  rows_pt = np_rows // NS
  mesh = plsc.VectorSubcoreMesh(core_axis_name="c", subcore_axis_name="s")

  @functools.partial(
      pl.kernel,
      out_type=jax.ShapeDtypeStruct((NC, np_rows), jnp.float32),
      mesh=mesh,
      scratch_types=[
          pltpu.VMEM_SHARED((np_rows,), jnp.float32),
          pltpu.VMEM((C, K), jnp.int32),
          pltpu.VMEM((K,), jnp.float32),
          pltpu.VMEM((rows_pt,), jnp.float32),
      ],
  )
  def deg_kernel(dst_hbm, out_hbm, deg_sh, idx_v, ones_v, zero_v):
    c = lax.axis_index("c")
    s = lax.axis_index("s")
    wid = c * NS + s
    for i in range(K // 16):
      ones_v[pl.ds(i * 16, 16)] = jnp.ones((16,), jnp.float32)
    for i in range(rows_pt // 16):
      zero_v[pl.ds(i * 16, 16)] = jnp.zeros((16,), jnp.float32)
    pltpu.sync_copy(zero_v, deg_sh.at[pl.ds(s * rows_pt, rows_pt)])
    pltpu.sync_copy(dst_hbm.at[wid], idx_v)
    plsc.subcore_barrier()

    def body(j, carry):
      pltpu.sync_copy(ones_v, deg_sh.at[idx_v.at[j]], add=True)
      return carry

    lax.fori_loop(0, C, body, 0, unroll=False)
    plsc.subcore_barrier()
    pltpu.sync_copy(deg_sh.at[pl.ds(s * rows_pt, rows_pt)],
                    out_hbm.at[c, pl.ds(s * rows_pt, rows_pt)])

  return deg_kernel(dst_r)


def _sc_scatter(hp, src_r, dst_r, zeros2d, np_rows, d):
  C = src_r.shape[1]
  rows_pt = np_rows // NS
  mesh = plsc.VectorSubcoreMesh(core_axis_name="c", subcore_axis_name="s")

  @functools.partial(
      pl.kernel,
      out_type=jax.ShapeDtypeStruct((NC, np_rows, d), jnp.float32),
      mesh=mesh,
      scratch_types=[
          pltpu.VMEM_SHARED((np_rows, d), jnp.float32),
          pltpu.VMEM((C, K), jnp.int32),
          pltpu.VMEM((4, K), jnp.int32),
          pltpu.VMEM((2, K, d), jnp.float32),
          pltpu.SemaphoreType.DMA,
          pltpu.SemaphoreType.DMA,
          pltpu.SemaphoreType.DMA,
          pltpu.SemaphoreType.DMA,
          [pltpu.SemaphoreType.DMA] * 4,
      ],
  )
  def scat_kernel(hp_hbm, src_hbm, dst_hbm, zero_hbm, out_hbm,
                  acc_sh, src_v, didx_v, rows_v, g0, g1, s0, s1, dsems):
    c = lax.axis_index("c")
    s = lax.axis_index("s")
    wid = c * NS + s
    row_slice = pl.ds(s * rows_pt, rows_pt)

    @pl.when(c == 0)
    def _():
      pltpu.sync_copy(hp_hbm.at[row_slice], acc_sh.at[row_slice])

    @pl.when(c != 0)
    def _():
      pltpu.sync_copy(zero_hbm.at[row_slice], acc_sh.at[row_slice])

    pltpu.sync_copy(src_hbm.at[wid], src_v)
    plsc.subcore_barrier()

    gsems = (g0, g1)
    ssems = (s0, s1)

    def wait_gather(p):
      pltpu.make_async_copy(
          hp_hbm.at[src_v.at[0]], rows_v.at[p], gsems[p]).wait()

    def wait_scatter(p):
      pltpu.make_async_copy(
          rows_v.at[0], acc_sh.at[didx_v.at[0]], ssems[p]).wait()

    def wait_didx(q):
      pltpu.make_async_copy(
          dst_hbm.at[wid, 0], didx_v.at[q], dsems[q]).wait()

    for q in range(3):
      pltpu.async_copy(dst_hbm.at[wid, q], didx_v.at[q], dsems[q])
    pltpu.async_copy(hp_hbm.at[src_v.at[0]], rows_v.at[0], g0)

    @pl.loop(0, C, step=4)
    def _(j):
      for u in range(4):
        jj = j + u
        p = u % 2
        if u == 0:
          @pl.when(j > 0)
          def _():
            wait_scatter(1)
        else:
          wait_scatter(1 - p)
        nxt = jj + 1
        if u < 3:
          pltpu.async_copy(hp_hbm.at[src_v.at[nxt]], rows_v.at[1 - p],
                           gsems[1 - p])
        else:
          @pl.when(nxt < C)
          def _():
            pltpu.async_copy(hp_hbm.at[src_v.at[nxt]], rows_v.at[1 - p],
                             gsems[1 - p])
        nid = jj + 3

        @pl.when(nid < C)
        def _():
          pltpu.async_copy(dst_hbm.at[wid, nid], didx_v.at[(u + 3) % 4],
                           dsems[(u + 3) % 4])

        wait_gather(p)
        wait_didx(u)
        pltpu.async_copy(rows_v.at[p], acc_sh.at[didx_v.at[u]], ssems[p],
                         add=True)

    wait_scatter((C - 1) % 2)
    plsc.subcore_barrier()
    pltpu.sync_copy(acc_sh.at[row_slice], out_hbm.at[c, row_slice])

  return scat_kernel(hp, src_r, dst_r, zeros2d)


def _tc_dinv(deg2, n, np_rows):
  blocks = np_rows // 128
  deg_r = deg2.reshape(NC, blocks, 128)

  def body(deg_ref, out_ref):
    deg = deg_ref[0] + deg_ref[1] + 1.0
    rid = (lax.broadcasted_iota(jnp.int32, (blocks, 128), 0) * 128
           + lax.broadcasted_iota(jnp.int32, (blocks, 128), 1))
    out_ref[...] = jnp.where(rid < n, lax.rsqrt(deg), 0.0)

  out = pl.pallas_call(
      body,
      out_shape=jax.ShapeDtypeStruct((blocks, 128), jnp.float32),
  )(deg_r)
  return out.reshape(np_rows, 1)


def _tc_mm_scale(x, w, dinv, np_rows, d):
  def body(x_ref, w_ref, dinv_ref, out_ref):
    out_ref[...] = jnp.dot(
        x_ref[...], w_ref[...], preferred_element_type=jnp.float32
    ) * dinv_ref[...]

  return pl.pallas_call(
      body,
      grid=(np_rows // 128,),
      in_specs=[
          pl.BlockSpec((128, d), lambda i: (i, 0)),
          pl.BlockSpec((d, d), lambda i: (0, 0)),
          pl.BlockSpec((128, 1), lambda i: (i, 0)),
      ],
      out_specs=pl.BlockSpec((128, d), lambda i: (i, 0)),
      out_shape=jax.ShapeDtypeStruct((np_rows, d), jnp.float32),
  )(x, w, dinv)


def _tc_fused_next(acc, dinv, b, w_next, np_rows, d):
  def body(acc_ref, dinv_ref, b_ref, w_ref, out_ref):
    v = dinv_ref[...] * (acc_ref[0] + acc_ref[1]) + b_ref[...]
    v = jnp.maximum(v, 0.0)
    out_ref[...] = jnp.dot(
        v, w_ref[...], preferred_element_type=jnp.float32
    ) * dinv_ref[...]

  return pl.pallas_call(
      body,
      grid=(np_rows // 128,),
      in_specs=[
          pl.BlockSpec((NC, 128, d), lambda i: (0, i, 0)),
          pl.BlockSpec((128, 1), lambda i: (i, 0)),
          pl.BlockSpec((1, d), lambda i: (0, 0)),
          pl.BlockSpec((d, d), lambda i: (0, 0)),
      ],
      out_specs=pl.BlockSpec((128, d), lambda i: (i, 0)),
      out_shape=jax.ShapeDtypeStruct((np_rows, d), jnp.float32),
  )(acc, dinv, b.reshape(1, d), w_next)


def _tc_final(acc, dinv, b, n, d):
  def body(acc_ref, dinv_ref, b_ref, out_ref):
    v = dinv_ref[...] * (acc_ref[0] + acc_ref[1]) + b_ref[...]
    ss = jnp.sum(v * v, axis=-1, keepdims=True)
    out_ref[...] = v * lax.rsqrt(jnp.maximum(ss, 1e-24))

  grid = (n + 127) // 128
  return pl.pallas_call(
      body,
      grid=(grid,),
      in_specs=[
          pl.BlockSpec((NC, 128, d), lambda i: (0, i, 0)),
          pl.BlockSpec((128, 1), lambda i: (i, 0)),
          pl.BlockSpec((1, d), lambda i: (0, 0)),
      ],
      out_specs=pl.BlockSpec((128, d), lambda i: (i, 0)),
      out_shape=jax.ShapeDtypeStruct((n, d), jnp.float32),
  )(acc, dinv, b.reshape(1, d))


@jax.jit
def kernel(x, edge_index, W1, b1, W2, b2, W3, b3):
  n, d = x.shape
  e = edge_index.shape[1]
  dummy = n
  np_rows = ((n + 1 + NS * 128 - 1) // (NS * 128)) * (NS * 128)

  src = edge_index[0].astype(jnp.int32)
  dst = edge_index[1].astype(jnp.int32)
  C = (e + NW * K - 1) // (NW * K)
  C = ((C + 3) // 4) * 4
  e2 = NW * C * K
  pad = e2 - e
  fill = dummy + (jnp.arange(pad, dtype=jnp.int32) % (np_rows - n))
  src_r = jnp.concatenate([src, fill]).reshape(NW, C, K)
  dst_r = jnp.concatenate([dst, fill]).reshape(NW, C, K)

  x_pad = jnp.zeros((np_rows, d), jnp.float32).at[:n].set(x)
  zeros2d = jnp.zeros((np_rows, d), jnp.float32)

  deg2 = _sc_degree(dst_r, np_rows)
  dinv = _tc_dinv(deg2, n, np_rows)

  hp1 = _tc_mm_scale(x_pad, W1, dinv, np_rows, d)
  acc1 = _sc_scatter(hp1, src_r, dst_r, zeros2d, np_rows, d)
  hp2 = _tc_fused_next(acc1, dinv, b1, W2, np_rows, d)
  acc2 = _sc_scatter(hp2, src_r, dst_r, zeros2d, np_rows, d)
  hp3 = _tc_fused_next(acc2, dinv, b2, W3, np_rows, d)
  acc3 = _sc_scatter(hp3, src_r, dst_r, zeros2d, np_rows, d)
  return _tc_final(acc3, dinv, b3, n, d)
